# baseline (speedup 1.0000x reference)
import jax
import jax.numpy as jnp
from jax.experimental import pallas as pl
from jax.experimental.pallas import tpu as pltpu

BM = 128


def _mm_kernel(a_ref, c_ref, w_ref, b_ref, o_ref):
    o_ref[0] = (
        jnp.dot(a_ref[...], w_ref[...], preferred_element_type=jnp.float32)
        + b_ref[...]
    )
    o_ref[1] = (
        jnp.dot(c_ref[...], w_ref[...], preferred_element_type=jnp.float32)
        + b_ref[...]
    )


@jax.jit
def kernel(adj, W, b):
    n, k = adj.shape
    out_f = W.shape[1]
    b2 = b.reshape(1, out_f)
    half = n // 2
    grid = (half // BM,)
    off = half // BM
    out3 = pl.pallas_call(
        _mm_kernel,
        grid=grid,
        in_specs=[
            pl.BlockSpec((BM, k), lambda i: (i, 0)),
            pl.BlockSpec((BM, k), lambda i: (i + off, 0)),
            pl.BlockSpec((k, out_f), lambda i: (0, 0)),
            pl.BlockSpec((1, out_f), lambda i: (0, 0)),
        ],
        out_specs=pl.BlockSpec((2, BM, out_f), lambda i: (0, i, 0)),
        out_shape=jax.ShapeDtypeStruct((2, half, out_f), jnp.float32),
        compiler_params=pltpu.CompilerParams(
            dimension_semantics=("arbitrary",),
        ),
    )(adj, adj, W, b2)
    return out3.reshape(n, out_f)


# speedup vs baseline: 1.0075x; 1.0075x over previous
import jax
import jax.numpy as jnp
from jax.experimental import pallas as pl
from jax.experimental.pallas import tpu as pltpu

BM = 256


def _mm_kernel(adj_ref, w_ref, b_ref, o_ref):
    a16 = adj_ref[...].astype(jnp.bfloat16)
    o_ref[...] = (
        jnp.dot(a16, w_ref[...], preferred_element_type=jnp.float32)
        + b_ref[...]
    )


@jax.jit
def kernel(adj, W, b):
    n, k = adj.shape
    out_f = W.shape[1]
    b2 = b.reshape(1, out_f)
    w16 = W.astype(jnp.bfloat16)
    return pl.pallas_call(
        _mm_kernel,
        grid=(n // BM,),
        in_specs=[
            pl.BlockSpec((BM, k), lambda i: (i, 0)),
            pl.BlockSpec((k, out_f), lambda i: (0, 0)),
            pl.BlockSpec((1, out_f), lambda i: (0, 0)),
        ],
        out_specs=pl.BlockSpec((BM, out_f), lambda i: (i, 0)),
        out_shape=jax.ShapeDtypeStruct((n, out_f), jnp.float32),
        compiler_params=pltpu.CompilerParams(
            dimension_semantics=("arbitrary",),
        ),
    )(adj, w16, b2)
